# baseline (speedup 1.0000x reference)
import jax
import jax.numpy as jnp
from jax import lax
from jax.experimental import pallas as pl
from jax.experimental.pallas import tpu as pltpu
from jax.experimental.pallas import tpu_sc as plsc

VOCAB = 1000000
MAXLEN = 200
EMBED = 32
BATCH = 4096

NC = 2
NS = 16
NW = NC * NS
LT = MAXLEN // 8
BT = BATCH // 128
ET = EMBED // 8


def _body(x_hbm, tok_hbm, pos_hbm, out_hbm,
          idx_v, gbuf0, gbuf1, obuf0, obuf1, pos_v,
          sg0, sg1, ss0, ss1, si):
    gbuf = (gbuf0, gbuf1)
    obuf = (obuf0, obuf1)
    sg = (sg0, sg1)
    ss = (ss0, ss1)
    w = lax.axis_index("s") * NC + lax.axis_index("c")

    pltpu.sync_copy(pos_hbm, pos_v)
    for lt in range(LT):
        pltpu.async_copy(x_hbm.at[lt, w], idx_v.at[lt], si)
    pltpu.make_async_copy(x_hbm.at[0, 0], idx_v, si).wait()

    iota = lax.iota(jnp.int32, 16)
    pat = (iota * 128, iota * 128 + 2048)

    def fire_gather(l, b):
        lt = l // 8
        ls = l % 8
        pltpu.async_copy(tok_hbm.at[idx_v.at[lt, ls]], gbuf[b], sg[b])

    fire_gather(0, 0)

    @pl.loop(0, MAXLEN, step=2)
    def _seq(g):
        for b in range(2):
            l = g + b
            o = 1 - b

            @pl.when(l + 1 < MAXLEN)
            def _():
                fire_gather(l + 1, o)

            pltpu.make_async_copy(tok_hbm.at[pl.ds(0, 128)], gbuf[b], sg[b]).wait()

            @pl.when(l >= 2)
            def _():
                for _ in range(ET):
                    pltpu.make_async_copy(
                        obuf[b].at[pl.ds(0, 1024)], out_hbm.at[0, 0, 0], ss[b]).wait()

            ph = (pos_v.at[l, pl.ds(0, 16)][...], pos_v.at[l, pl.ds(16, 16)][...])

            @pl.loop(0, 128, unroll=4)
            def _row(bb):
                for h in range(2):
                    v = gbuf[b].at[bb, pl.ds(h * 16, 16)][...] + ph[h]
                    plsc.store_scatter(obuf[b], [pat[h] + bb], v)

            for et in range(ET):
                pltpu.async_copy(
                    obuf[b].at[pl.ds(et * 1024, 1024)], out_hbm.at[l, et, w], ss[b])

    for b in range(2):
        for _ in range(ET):
            pltpu.make_async_copy(
                obuf[b].at[pl.ds(0, 1024)], out_hbm.at[0, 0, 0], ss[b]).wait()


def kernel(x, token_table, pos_table):
    x4 = jnp.transpose(
        jnp.reshape(jnp.transpose(x), (LT, 8, BT, 128)), (0, 2, 1, 3))
    mesh = plsc.VectorSubcoreMesh(core_axis_name="c", subcore_axis_name="s")
    k = pl.kernel(
        _body,
        out_type=jax.ShapeDtypeStruct((MAXLEN, ET, BT, 1024), jnp.float32),
        mesh=mesh,
        compiler_params=pltpu.CompilerParams(
            use_tc_tiling_on_sc=False, needs_layout_passes=False),
        scratch_types=[
            pltpu.VMEM((LT, 8, 128), jnp.int32),
            pltpu.VMEM((128, EMBED), jnp.float32),
            pltpu.VMEM((128, EMBED), jnp.float32),
            pltpu.VMEM((ET * 1024,), jnp.float32),
            pltpu.VMEM((ET * 1024,), jnp.float32),
            pltpu.VMEM((MAXLEN, EMBED), jnp.float32),
            pltpu.SemaphoreType.DMA,
            pltpu.SemaphoreType.DMA,
            pltpu.SemaphoreType.DMA,
            pltpu.SemaphoreType.DMA,
            pltpu.SemaphoreType.DMA,
        ],
    )
    out5 = k(x4, token_table, pos_table)
    return jnp.reshape(
        jnp.transpose(jnp.reshape(out5, (MAXLEN, ET, BT, 8, 128)),
                      (2, 4, 0, 1, 3)),
        (BATCH, MAXLEN, EMBED))
